# baseline (device time: 100940 ns/iter reference)
import jax
import jax.numpy as jnp
from jax import lax
from jax.experimental import pallas as pl
from jax.experimental.pallas import tpu as pltpu

CHUNKS = 16
SLOTS = 4


def kernel(x):
    m, n = x.shape
    half = n // 2
    cr = m // CHUNKS

    def body(x_ref, out_ref, f32_buf, keep_buf, send_buf,
             in_sems, send_sems, local_sems, recv_sems):
        my_x = lax.axis_index("x")
        my_y = lax.axis_index("y")
        my_z = lax.axis_index("z")
        other_y = 1 - my_y
        partner = (my_x, other_y, my_z)

        def make_load(k):
            return pltpu.make_async_copy(
                x_ref.at[pl.ds(k * cr, cr), :],
                f32_buf.at[k % SLOTS],
                in_sems.at[k % SLOTS],
            )

        def make_local(k):
            return pltpu.make_async_copy(
                keep_buf.at[k % SLOTS],
                out_ref.at[pl.ds(my_y * m + k * cr, cr), :],
                local_sems.at[k % SLOTS],
            )

        def make_rdma(k):
            return pltpu.make_async_remote_copy(
                src_ref=send_buf.at[k % SLOTS],
                dst_ref=out_ref.at[pl.ds(my_y * m + k * cr, cr), :],
                send_sem=send_sems.at[k % SLOTS],
                recv_sem=recv_sems.at[k],
                device_id=partner,
                device_id_type=pl.DeviceIdType.MESH,
            )

        loads = [make_load(k) for k in range(CHUNKS)]
        locals_ = [make_local(k) for k in range(CHUNKS)]
        rdmas = [make_rdma(k) for k in range(CHUNKS)]

        for k in range(min(SLOTS, CHUNKS)):
            loads[k].start()

        barrier_sem = pltpu.get_barrier_semaphore()
        pl.semaphore_signal(
            barrier_sem, inc=1,
            device_id=partner, device_id_type=pl.DeviceIdType.MESH,
        )
        pl.semaphore_wait(barrier_sem, 1)

        for k in range(CHUNKS):
            s = k % SLOTS
            loads[k].wait()
            if k >= SLOTS:
                rdmas[k - SLOTS].wait_send()
                locals_[k - SLOTS].wait()
            @pl.when(my_y == 0)
            def _():
                keep_buf[s] = f32_buf[s][:, :half].astype(jnp.bfloat16)
                send_buf[s] = f32_buf[s][:, half:].astype(jnp.bfloat16)

            @pl.when(my_y == 1)
            def _():
                keep_buf[s] = f32_buf[s][:, half:].astype(jnp.bfloat16)
                send_buf[s] = f32_buf[s][:, :half].astype(jnp.bfloat16)
            if k + SLOTS < CHUNKS:
                loads[k + SLOTS].start()
            rdmas[k].start()
            locals_[k].start()

        for k in range(max(CHUNKS - SLOTS, 0), CHUNKS):
            rdmas[k].wait_send()
            locals_[k].wait()
        for k in range(CHUNKS):
            rdmas[k].wait_recv()

    return pl.pallas_call(
        body,
        out_shape=jax.ShapeDtypeStruct((2 * m, half), jnp.bfloat16),
        in_specs=[pl.BlockSpec(memory_space=pl.ANY)],
        out_specs=pl.BlockSpec(memory_space=pl.ANY),
        scratch_shapes=[
            pltpu.VMEM((SLOTS, cr, n), jnp.float32),
            pltpu.VMEM((SLOTS, cr, half), jnp.bfloat16),
            pltpu.VMEM((SLOTS, cr, half), jnp.bfloat16),
            pltpu.SemaphoreType.DMA((SLOTS,)),
            pltpu.SemaphoreType.DMA((SLOTS,)),
            pltpu.SemaphoreType.DMA((SLOTS,)),
            pltpu.SemaphoreType.DMA((CHUNKS,)),
        ],
        compiler_params=pltpu.CompilerParams(collective_id=0),
    )(x)
